# baseline (speedup 1.0000x reference)
import jax
import jax.numpy as jnp
from jax.experimental import pallas as pl
from jax.experimental.pallas import tpu as pltpu

_N, _D, _G = 50000, 256, 128
_BLK = 12504
_NB = -(-_N // _BLK)
_NPAD = _NB * _BLK
_INV = 0.9999950000374996


def _stream(batch_ref, x_ref, vn_ref, h_ref, part_ref):
    i = pl.program_id(0)
    vn = vn_ref[0, :]
    hb = x_ref[...] + vn[None, :]
    h_ref[...] = hb
    ids = batch_ref[0, 0, :]
    valid = (jax.lax.broadcasted_iota(jnp.int32, (_BLK, 1), 0)
             + i * _BLK) < _N
    hb_m = jnp.where(valid, hb, 0.0)
    oh = (jax.lax.broadcasted_iota(jnp.int32, (_G, _BLK), 0)
          == ids[None, :]).astype(jnp.float32)
    part_ref[0] = jnp.dot(oh, hb_m, preferred_element_type=jnp.float32)


def _finalize(part_ref, vn_ref, w1_ref, b1_ref, g1_ref, be1_ref,
              w2_ref, b2_ref, g2_ref, be2_ref, t_ref):
    pooled = jnp.sum(part_ref[...], axis=0) + vn_ref[0, :][None, :]
    t = jnp.dot(pooled, w1_ref[...], preferred_element_type=jnp.float32)
    t = (t + b1_ref[0, :]) * (g1_ref[0, :] * _INV) + be1_ref[0, :]
    t = jnp.maximum(t, 0.0)
    t = jnp.dot(t, w2_ref[...], preferred_element_type=jnp.float32)
    t = (t + b2_ref[0, :]) * (g2_ref[0, :] * _INV) + be2_ref[0, :]
    t_ref[...] = jnp.maximum(t, 0.0)


def kernel(x, edge_index, batch, vn_w, w1, b1, g1, be1, w2, b2, g2, be2):
    del edge_index
    batch3 = jnp.pad(batch, (0, _NPAD - _N)).reshape(_NB, 1, _BLK)

    h, parts = pl.pallas_call(
        _stream,
        grid=(_NB,),
        in_specs=[
            pl.BlockSpec((1, 1, _BLK), lambda i: (i, 0, 0)),
            pl.BlockSpec((_BLK, _D), lambda i: (i, 0)),
            pl.BlockSpec((1, _D), lambda i: (0, 0)),
        ],
        out_specs=[
            pl.BlockSpec((_BLK, _D), lambda i: (i, 0)),
            pl.BlockSpec((1, _G, _D), lambda i: (i, 0, 0)),
        ],
        out_shape=[
            jax.ShapeDtypeStruct((_N, _D), jnp.float32),
            jax.ShapeDtypeStruct((_NB, _G, _D), jnp.float32),
        ],
        compiler_params=pltpu.CompilerParams(
            dimension_semantics=("parallel",),
        ),
    )(batch3, x, vn_w)

    t = pl.pallas_call(
        _finalize,
        out_shape=jax.ShapeDtypeStruct((_G, _D), jnp.float32),
    )(parts, vn_w, w1,
      b1.reshape(1, 2 * _D), g1.reshape(1, 2 * _D), be1.reshape(1, 2 * _D),
      w2, b2.reshape(1, _D), g2.reshape(1, _D), be2.reshape(1, _D))
    return (h, t)


# speedup vs baseline: 1.0164x; 1.0164x over previous
import jax
import jax.numpy as jnp
from jax.experimental import pallas as pl
from jax.experimental.pallas import tpu as pltpu

_N, _D, _G = 50000, 256, 128
_BLK = 13504
_NB = -(-_N // _BLK)
_NPAD = _NB * _BLK
_INV = 0.9999950000374996


def _stream(batch_ref, x_ref, vn_ref, h_ref, part_ref):
    i = pl.program_id(0)
    vn = vn_ref[0, :]
    hb = x_ref[...] + vn[None, :]
    h_ref[...] = hb
    ids = batch_ref[0, 0, :]
    valid = (jax.lax.broadcasted_iota(jnp.int32, (_BLK, 1), 0)
             + i * _BLK) < _N
    hb_m = jnp.where(valid, hb, 0.0)
    oh = (jax.lax.broadcasted_iota(jnp.int32, (_G, _BLK), 0)
          == ids[None, :]).astype(jnp.float32)
    part_ref[0] = jnp.dot(oh, hb_m, preferred_element_type=jnp.float32)


def _finalize(part_ref, vn_ref, w1_ref, b1_ref, g1_ref, be1_ref,
              w2_ref, b2_ref, g2_ref, be2_ref, t_ref):
    pooled = jnp.sum(part_ref[...], axis=0) + vn_ref[0, :][None, :]
    t = jnp.dot(pooled, w1_ref[...], preferred_element_type=jnp.float32)
    t = (t + b1_ref[0, :]) * (g1_ref[0, :] * _INV) + be1_ref[0, :]
    t = jnp.maximum(t, 0.0)
    t = jnp.dot(t, w2_ref[...], preferred_element_type=jnp.float32)
    t = (t + b2_ref[0, :]) * (g2_ref[0, :] * _INV) + be2_ref[0, :]
    t_ref[...] = jnp.maximum(t, 0.0)


def kernel(x, edge_index, batch, vn_w, w1, b1, g1, be1, w2, b2, g2, be2):
    del edge_index
    batch3 = jnp.pad(batch, (0, _NPAD - _N)).reshape(_NB, 1, _BLK)

    h, parts = pl.pallas_call(
        _stream,
        grid=(_NB,),
        in_specs=[
            pl.BlockSpec((1, 1, _BLK), lambda i: (i, 0, 0)),
            pl.BlockSpec((_BLK, _D), lambda i: (i, 0)),
            pl.BlockSpec((1, _D), lambda i: (0, 0)),
        ],
        out_specs=[
            pl.BlockSpec((_BLK, _D), lambda i: (i, 0)),
            pl.BlockSpec((1, _G, _D), lambda i: (i, 0, 0)),
        ],
        out_shape=[
            jax.ShapeDtypeStruct((_N, _D), jnp.float32),
            jax.ShapeDtypeStruct((_NB, _G, _D), jnp.float32),
        ],
        compiler_params=pltpu.CompilerParams(
            dimension_semantics=("parallel",),
        ),
    )(batch3, x, vn_w)

    t = pl.pallas_call(
        _finalize,
        out_shape=jax.ShapeDtypeStruct((_G, _D), jnp.float32),
    )(parts, vn_w, w1,
      b1.reshape(1, 2 * _D), g1.reshape(1, 2 * _D), be1.reshape(1, 2 * _D),
      w2, b2.reshape(1, _D), g2.reshape(1, _D), be2.reshape(1, _D))
    return (h, t)


# speedup vs baseline: 1.0188x; 1.0024x over previous
import jax
import jax.numpy as jnp
from jax.experimental import pallas as pl
from jax.experimental.pallas import tpu as pltpu

_N, _D, _G = 50000, 256, 128
_BLK = 14504
_NB = -(-_N // _BLK)
_NPAD = _NB * _BLK
_INV = 0.9999950000374996


def _stream(batch_ref, x_ref, vn_ref, h_ref, part_ref):
    i = pl.program_id(0)
    vn = vn_ref[0, :]
    hb = x_ref[...] + vn[None, :]
    h_ref[...] = hb
    ids = batch_ref[0, 0, :]
    valid = (jax.lax.broadcasted_iota(jnp.int32, (_BLK, 1), 0)
             + i * _BLK) < _N
    hb_m = jnp.where(valid, hb, 0.0)
    oh = (jax.lax.broadcasted_iota(jnp.int32, (_G, _BLK), 0)
          == ids[None, :]).astype(jnp.float32)
    part_ref[0] = jnp.dot(oh, hb_m, preferred_element_type=jnp.float32)


def _finalize(part_ref, vn_ref, w1_ref, b1_ref, g1_ref, be1_ref,
              w2_ref, b2_ref, g2_ref, be2_ref, t_ref):
    pooled = jnp.sum(part_ref[...], axis=0) + vn_ref[0, :][None, :]
    t = jnp.dot(pooled, w1_ref[...], preferred_element_type=jnp.float32)
    t = (t + b1_ref[0, :]) * (g1_ref[0, :] * _INV) + be1_ref[0, :]
    t = jnp.maximum(t, 0.0)
    t = jnp.dot(t, w2_ref[...], preferred_element_type=jnp.float32)
    t = (t + b2_ref[0, :]) * (g2_ref[0, :] * _INV) + be2_ref[0, :]
    t_ref[...] = jnp.maximum(t, 0.0)


def kernel(x, edge_index, batch, vn_w, w1, b1, g1, be1, w2, b2, g2, be2):
    del edge_index
    batch3 = jnp.pad(batch, (0, _NPAD - _N)).reshape(_NB, 1, _BLK)

    h, parts = pl.pallas_call(
        _stream,
        grid=(_NB,),
        in_specs=[
            pl.BlockSpec((1, 1, _BLK), lambda i: (i, 0, 0)),
            pl.BlockSpec((_BLK, _D), lambda i: (i, 0)),
            pl.BlockSpec((1, _D), lambda i: (0, 0)),
        ],
        out_specs=[
            pl.BlockSpec((_BLK, _D), lambda i: (i, 0)),
            pl.BlockSpec((1, _G, _D), lambda i: (i, 0, 0)),
        ],
        out_shape=[
            jax.ShapeDtypeStruct((_N, _D), jnp.float32),
            jax.ShapeDtypeStruct((_NB, _G, _D), jnp.float32),
        ],
        compiler_params=pltpu.CompilerParams(
            dimension_semantics=("parallel",),
        ),
    )(batch3, x, vn_w)

    t = pl.pallas_call(
        _finalize,
        out_shape=jax.ShapeDtypeStruct((_G, _D), jnp.float32),
    )(parts, vn_w, w1,
      b1.reshape(1, 2 * _D), g1.reshape(1, 2 * _D), be1.reshape(1, 2 * _D),
      w2, b2.reshape(1, _D), g2.reshape(1, _D), be2.reshape(1, _D))
    return (h, t)
